# baseline (speedup 1.0000x reference)
import jax
import jax.numpy as jnp
from jax.experimental import pallas as pl
from jax.experimental.pallas import tpu as pltpu

NUM_CLUSTERS = 1024
CODE_DIM = 256
BLOCK_N = 1024
SPLIT = 4
SUB_N = BLOCK_N // SPLIT
REFINE_SLOTS = 12
GAP_THRESHOLD = 3e-4


def _ref_style_dist(x, c):
    diff = x - c
    sq = diff * diff
    t = sq[:, :128] + sq[:, 128:]
    acc = t
    for i in range(1, 16):
        acc = acc + jnp.roll(t, -8 * i, axis=1)
    a = acc + jnp.roll(acc, -4, axis=1)
    b = a + jnp.roll(a, -2, axis=1)
    d = b + jnp.roll(b, -1, axis=1)
    return d[:, 0:1]


def _assign_kernel(x_ref, c_ref, out_ref, cn_ref):
    c = c_ref[...]

    @pl.when(pl.program_id(0) == 0)
    def _():
        cn_ref[0:1, :] = jnp.sum(c * c, axis=1, keepdims=True).T

    cn = cn_ref[0:1, :]
    ambs = []
    for q in range(SPLIT):
        x = x_ref[q * SUB_N:(q + 1) * SUB_N, :]
        dots = jax.lax.dot_general(
            x + x, c,
            dimension_numbers=(((1,), (1,)), ((), ())),
            preferred_element_type=jnp.float32,
            precision=jax.lax.Precision.HIGHEST,
        )
        s = cn - dots
        minval = jnp.min(s, axis=1, keepdims=True)
        out_ref[q * SUB_N:(q + 1) * SUB_N, :] = (s == minval).astype(jnp.float32)
        near = (s < minval + GAP_THRESHOLD).astype(jnp.int32)
        ambs.append(jnp.sum(near, axis=1, keepdims=True) > 1)

    ambiguous = jnp.concatenate(ambs, axis=0)
    rowii = jax.lax.broadcasted_iota(jnp.int32, ambiguous.shape, 0)
    m = jnp.where(ambiguous, rowii, BLOCK_N)
    for _ in range(REFINE_SLOTS):
        r = jnp.min(m)

        @pl.when(r < BLOCK_N)
        def _():
            d = _ref_style_dist(x_ref[pl.ds(r, 1), :], c)
            dmin = jnp.min(d)
            kii = jax.lax.broadcasted_iota(jnp.int32, d.shape, 0)
            kidx = jnp.min(jnp.where(d == dmin, kii, NUM_CLUSTERS))
            oi = jax.lax.broadcasted_iota(jnp.int32, (1, NUM_CLUSTERS), 1)
            out_ref[pl.ds(r, 1), :] = (oi == kidx).astype(jnp.float32)

        m = jnp.where(rowii == r, BLOCK_N, m)


@jax.jit
def kernel(inputs, centroids):
    d = inputs.shape[-1]
    x = inputs.reshape(-1, d)
    n = x.shape[0]
    out = pl.pallas_call(
        _assign_kernel,
        grid=(n // BLOCK_N,),
        in_specs=[
            pl.BlockSpec((BLOCK_N, d), lambda i: (i, 0)),
            pl.BlockSpec((NUM_CLUSTERS, d), lambda i: (0, 0)),
        ],
        out_specs=pl.BlockSpec((BLOCK_N, NUM_CLUSTERS), lambda i: (i, 0)),
        out_shape=jax.ShapeDtypeStruct((n, NUM_CLUSTERS), jnp.float32),
        scratch_shapes=[pltpu.VMEM((8, NUM_CLUSTERS), jnp.float32)],
    )(x, centroids)
    return out.reshape(inputs.shape[:-1] + (NUM_CLUSTERS,))


# speedup vs baseline: 1.4568x; 1.4568x over previous
import jax
import jax.numpy as jnp
from jax.experimental import pallas as pl
from jax.experimental.pallas import tpu as pltpu

NUM_CLUSTERS = 1024
CODE_DIM = 256
BLOCK_N = 1024
REFINE_SLOTS = 12
GAP_THRESHOLD = 3e-4

_DOT_DIMS = (((1,), (1,)), ((), ()))


def _ref_style_dist(x, c):
    diff = x - c
    sq = diff * diff
    t = sq[:, :128] + sq[:, 128:]
    acc = t
    for i in range(1, 16):
        acc = acc + jnp.roll(t, -8 * i, axis=1)
    a = acc + jnp.roll(acc, -4, axis=1)
    b = a + jnp.roll(a, -2, axis=1)
    d = b + jnp.roll(b, -1, axis=1)
    return d[:, 0:1]


def _assign_kernel(x_ref, c_ref, out_ref, cn_ref, ch_ref, cl_ref):
    c = c_ref[...]

    @pl.when(pl.program_id(0) == 0)
    def _():
        cn_ref[0:1, :] = jnp.sum(c * c, axis=1, keepdims=True).T
        ch = c.astype(jnp.bfloat16)
        ch_ref[...] = ch
        cl_ref[...] = (c - ch.astype(jnp.float32)).astype(jnp.bfloat16)

    x2 = x_ref[...]
    x2 = x2 + x2
    xh = x2.astype(jnp.bfloat16)
    xl = (x2 - xh.astype(jnp.float32)).astype(jnp.bfloat16)
    ch = ch_ref[...]
    cl = cl_ref[...]

    def bdot(a, b):
        return jax.lax.dot_general(a, b, dimension_numbers=_DOT_DIMS,
                                   preferred_element_type=jnp.float32)

    dots = bdot(xh, ch) + (bdot(xh, cl) + bdot(xl, ch))
    s = cn_ref[0:1, :] - dots
    minval = jnp.min(s, axis=1, keepdims=True)
    out_ref[...] = (s == minval).astype(jnp.float32)
    near = (s < minval + GAP_THRESHOLD).astype(jnp.int32)
    count = jnp.sum(near, axis=1, keepdims=True)
    ambiguous = count > 1

    rowii = jax.lax.broadcasted_iota(jnp.int32, ambiguous.shape, 0)
    m = jnp.where(ambiguous, rowii, BLOCK_N)
    for _ in range(REFINE_SLOTS):
        r = jnp.min(m)

        @pl.when(r < BLOCK_N)
        def _():
            d = _ref_style_dist(x_ref[pl.ds(r, 1), :], c)
            dmin = jnp.min(d)
            kii = jax.lax.broadcasted_iota(jnp.int32, d.shape, 0)
            kidx = jnp.min(jnp.where(d == dmin, kii, NUM_CLUSTERS))
            oi = jax.lax.broadcasted_iota(jnp.int32, (1, NUM_CLUSTERS), 1)
            out_ref[pl.ds(r, 1), :] = (oi == kidx).astype(jnp.float32)

        m = jnp.where(rowii == r, BLOCK_N, m)


@jax.jit
def kernel(inputs, centroids):
    d = inputs.shape[-1]
    x = inputs.reshape(-1, d)
    n = x.shape[0]
    out = pl.pallas_call(
        _assign_kernel,
        grid=(n // BLOCK_N,),
        in_specs=[
            pl.BlockSpec((BLOCK_N, d), lambda i: (i, 0)),
            pl.BlockSpec((NUM_CLUSTERS, d), lambda i: (0, 0)),
        ],
        out_specs=pl.BlockSpec((BLOCK_N, NUM_CLUSTERS), lambda i: (i, 0)),
        out_shape=jax.ShapeDtypeStruct((n, NUM_CLUSTERS), jnp.float32),
        scratch_shapes=[
            pltpu.VMEM((8, NUM_CLUSTERS), jnp.float32),
            pltpu.VMEM((NUM_CLUSTERS, CODE_DIM), jnp.bfloat16),
            pltpu.VMEM((NUM_CLUSTERS, CODE_DIM), jnp.bfloat16),
        ],
    )(x, centroids)
    return out.reshape(inputs.shape[:-1] + (NUM_CLUSTERS,))


# speedup vs baseline: 1.5266x; 1.0479x over previous
import jax
import jax.numpy as jnp
from jax.experimental import pallas as pl
from jax.experimental.pallas import tpu as pltpu

NUM_CLUSTERS = 1024
CODE_DIM = 256
BLOCK_N = 2048
REFINE_SLOTS = 12
GAP_THRESHOLD = 3e-4

_DOT_DIMS = (((1,), (1,)), ((), ()))


def _ref_style_dist(x, c):
    diff = x - c
    sq = diff * diff
    t = sq[:, :128] + sq[:, 128:]
    acc = t
    for i in range(1, 16):
        acc = acc + jnp.roll(t, -8 * i, axis=1)
    a = acc + jnp.roll(acc, -4, axis=1)
    b = a + jnp.roll(a, -2, axis=1)
    d = b + jnp.roll(b, -1, axis=1)
    return d[:, 0:1]


def _assign_kernel(x_ref, c_ref, out_ref, cn_ref, ch_ref, cl_ref):
    c = c_ref[...]

    @pl.when(pl.program_id(0) == 0)
    def _():
        cn_ref[0:1, :] = jnp.sum(c * c, axis=1, keepdims=True).T
        ch = c.astype(jnp.bfloat16)
        ch_ref[...] = ch
        cl_ref[...] = (c - ch.astype(jnp.float32)).astype(jnp.bfloat16)

    x2 = x_ref[...]
    x2 = x2 + x2
    xh = x2.astype(jnp.bfloat16)
    xl = (x2 - xh.astype(jnp.float32)).astype(jnp.bfloat16)
    ch = ch_ref[...]
    cl = cl_ref[...]

    def bdot(a, b):
        return jax.lax.dot_general(a, b, dimension_numbers=_DOT_DIMS,
                                   preferred_element_type=jnp.float32)

    dots = bdot(xh, ch) + (bdot(xh, cl) + bdot(xl, ch))
    s = cn_ref[0:1, :] - dots
    minval = jnp.min(s, axis=1, keepdims=True)
    out_ref[...] = (s == minval).astype(jnp.float32)
    near = (s < minval + GAP_THRESHOLD).astype(jnp.int32)
    count = jnp.sum(near, axis=1, keepdims=True)
    ambiguous = count > 1

    rowii = jax.lax.broadcasted_iota(jnp.int32, ambiguous.shape, 0)
    m = jnp.where(ambiguous, rowii, BLOCK_N)
    for _ in range(REFINE_SLOTS):
        r = jnp.min(m)

        @pl.when(r < BLOCK_N)
        def _():
            d = _ref_style_dist(x_ref[pl.ds(r, 1), :], c)
            dmin = jnp.min(d)
            kii = jax.lax.broadcasted_iota(jnp.int32, d.shape, 0)
            kidx = jnp.min(jnp.where(d == dmin, kii, NUM_CLUSTERS))
            oi = jax.lax.broadcasted_iota(jnp.int32, (1, NUM_CLUSTERS), 1)
            out_ref[pl.ds(r, 1), :] = (oi == kidx).astype(jnp.float32)

        m = jnp.where(rowii == r, BLOCK_N, m)


@jax.jit
def kernel(inputs, centroids):
    d = inputs.shape[-1]
    x = inputs.reshape(-1, d)
    n = x.shape[0]
    out = pl.pallas_call(
        _assign_kernel,
        grid=(n // BLOCK_N,),
        in_specs=[
            pl.BlockSpec((BLOCK_N, d), lambda i: (i, 0)),
            pl.BlockSpec((NUM_CLUSTERS, d), lambda i: (0, 0)),
        ],
        out_specs=pl.BlockSpec((BLOCK_N, NUM_CLUSTERS), lambda i: (i, 0)),
        out_shape=jax.ShapeDtypeStruct((n, NUM_CLUSTERS), jnp.float32),
        scratch_shapes=[
            pltpu.VMEM((8, NUM_CLUSTERS), jnp.float32),
            pltpu.VMEM((NUM_CLUSTERS, CODE_DIM), jnp.bfloat16),
            pltpu.VMEM((NUM_CLUSTERS, CODE_DIM), jnp.bfloat16),
        ],
    )(x, centroids)
    return out.reshape(inputs.shape[:-1] + (NUM_CLUSTERS,))
